# Initial kernel scaffold; baseline (speedup 1.0000x reference)
import jax
import jax.numpy as jnp
from jax.experimental import pallas as pl


def kernel(x, table, W, b):
    raise NotImplementedError("write your pallas kernel here")


# speedup vs baseline: 2.2659x; 2.2659x over previous
import functools

import jax
import jax.numpy as jnp
from jax import lax
from jax.experimental import pallas as pl
from jax.experimental.pallas import tpu as pltpu
from jax.experimental.pallas import tpu_sc as plsc


_ROWS_PER_BLOCK = 25000


def _score_body(table_ref, w_ref, b_ref, s_ref):
    s_ref[...] = (
        jnp.dot(table_ref[...], w_ref[...], preferred_element_type=jnp.float32)
        + b_ref[...]
    )


def _scores(table, W, b):
    n, h = table.shape
    grid = n // _ROWS_PER_BLOCK
    return pl.pallas_call(
        _score_body,
        grid=(grid,),
        in_specs=[
            pl.BlockSpec((_ROWS_PER_BLOCK, h), lambda i: (i, 0)),
            pl.BlockSpec((h, 1), lambda i: (0, 0)),
            pl.BlockSpec((1,), lambda i: (0,)),
        ],
        out_specs=pl.BlockSpec((_ROWS_PER_BLOCK, 1), lambda i: (i, 0)),
        out_shape=jax.ShapeDtypeStruct((n, 1), jnp.float32),
    )(table, W, b)


@functools.lru_cache(maxsize=None)
def _make_gather(n_out: int):
    info = plsc.get_sparse_core_info()
    nc, ns = info.num_cores, info.num_subcores
    nw = nc * ns
    assert n_out % nw == 0
    n_per_w = n_out // nw

    mesh = plsc.VectorSubcoreMesh(core_axis_name="c", subcore_axis_name="s")

    @functools.partial(
        pl.kernel,
        mesh=mesh,
        out_type=jax.ShapeDtypeStruct((n_out,), jnp.float32),
        scratch_types=[
            pltpu.VMEM((n_per_w,), jnp.int32),
            pltpu.VMEM((n_per_w,), jnp.float32),
            pltpu.SemaphoreType.DMA,
        ],
    )
    def gather_k(s_hbm, idx_hbm, out_hbm, idx_v, val_v, sem):
        wid = lax.axis_index("s") * nc + lax.axis_index("c")
        base = wid * n_per_w
        pltpu.sync_copy(idx_hbm.at[pl.ds(base, n_per_w)], idx_v)
        pltpu.async_copy(s_hbm.at[idx_v], val_v, sem).wait()
        pltpu.sync_copy(val_v, out_hbm.at[pl.ds(base, n_per_w)])

    return gather_k


def kernel(x, table, W, b):
    bsz, hist = x.shape
    n_out = bsz * hist
    s = _scores(table, W, b).reshape(-1)
    idx = x.reshape(-1).astype(jnp.int32)
    y = _make_gather(n_out)(s, idx)
    return y.reshape(bsz, hist, 1)


# speedup vs baseline: 2.6283x; 1.1599x over previous
import functools

import jax
import jax.numpy as jnp
from jax import lax
from jax.experimental import pallas as pl
from jax.experimental.pallas import tpu as pltpu
from jax.experimental.pallas import tpu_sc as plsc


_PACK = 64
_ROWS_PER_BLOCK = 1024


def _score_body(table_ref, w_ref, b_ref, s_ref):
    s_ref[...] = (
        jnp.dot(table_ref[...], w_ref[...], preferred_element_type=jnp.float32)
        + b_ref[...]
    )


def _scores(table, W, b):
    n, h = table.shape
    t4 = table.reshape(n // _PACK, h * _PACK)
    wk = jnp.kron(jnp.eye(_PACK, dtype=W.dtype), W)
    grid = -(-(n // _PACK) // _ROWS_PER_BLOCK)
    return pl.pallas_call(
        _score_body,
        grid=(grid,),
        in_specs=[
            pl.BlockSpec((_ROWS_PER_BLOCK, h * _PACK), lambda i: (i, 0)),
            pl.BlockSpec((h * _PACK, _PACK), lambda i: (0, 0)),
            pl.BlockSpec((1,), lambda i: (0,)),
        ],
        out_specs=pl.BlockSpec((_ROWS_PER_BLOCK, _PACK), lambda i: (i, 0)),
        out_shape=jax.ShapeDtypeStruct((n // _PACK, _PACK), jnp.float32),
    )(t4, wk, b)


@functools.lru_cache(maxsize=None)
def _make_gather(n_out: int):
    info = plsc.get_sparse_core_info()
    nc, ns = info.num_cores, info.num_subcores
    nw = nc * ns
    assert n_out % nw == 0
    n_per_w = n_out // nw

    mesh = plsc.VectorSubcoreMesh(core_axis_name="c", subcore_axis_name="s")

    @functools.partial(
        pl.kernel,
        mesh=mesh,
        out_type=jax.ShapeDtypeStruct((n_out,), jnp.float32),
        scratch_types=[
            pltpu.VMEM((n_per_w,), jnp.int32),
            pltpu.VMEM((n_per_w,), jnp.float32),
            pltpu.SemaphoreType.DMA,
        ],
    )
    def gather_k(s_hbm, idx_hbm, out_hbm, idx_v, val_v, sem):
        wid = lax.axis_index("s") * nc + lax.axis_index("c")
        base = wid * n_per_w
        pltpu.sync_copy(idx_hbm.at[pl.ds(base, n_per_w)], idx_v)
        pltpu.async_copy(s_hbm.at[idx_v], val_v, sem).wait()
        pltpu.sync_copy(val_v, out_hbm.at[pl.ds(base, n_per_w)])

    return gather_k


def kernel(x, table, W, b):
    bsz, hist = x.shape
    n_out = bsz * hist
    s = _scores(table, W, b).reshape(-1)
    idx = x.reshape(-1).astype(jnp.int32)
    y = _make_gather(n_out)(s, idx)
    return y.reshape(bsz, hist, 1)


# speedup vs baseline: 2.6546x; 1.0100x over previous
import functools

import jax
import jax.numpy as jnp
from jax import lax
from jax.experimental import pallas as pl
from jax.experimental.pallas import tpu as pltpu
from jax.experimental.pallas import tpu_sc as plsc


_PACK = 64
_ROWS_PER_BLOCK = 1024


def _score_body(table_ref, w_ref, b_ref, s_ref):
    blk = jnp.dot(
        table_ref[...], w_ref[...], preferred_element_type=jnp.float32
    ) + b_ref[...]
    b3 = blk.reshape(_ROWS_PER_BLOCK // 2, 2, _PACK)
    s_ref[...] = jnp.concatenate([b3[:, 0, :], b3[:, 1, :]], axis=1)


def _scores(table, W, b):
    n, h = table.shape
    t4 = table.reshape(n // _PACK, h * _PACK)
    wk = jnp.kron(jnp.eye(_PACK, dtype=W.dtype), W)
    grid = -(-(n // _PACK) // _ROWS_PER_BLOCK)
    out_rows = grid * _ROWS_PER_BLOCK // 2
    return pl.pallas_call(
        _score_body,
        grid=(grid,),
        in_specs=[
            pl.BlockSpec((_ROWS_PER_BLOCK, h * _PACK), lambda i: (i, 0)),
            pl.BlockSpec((h * _PACK, _PACK), lambda i: (0, 0)),
            pl.BlockSpec((1,), lambda i: (0,)),
        ],
        out_specs=pl.BlockSpec((_ROWS_PER_BLOCK // 2, 2 * _PACK), lambda i: (i, 0)),
        out_shape=jax.ShapeDtypeStruct((out_rows, 2 * _PACK), jnp.float32),
    )(t4, wk, b)


@functools.lru_cache(maxsize=None)
def _make_gather(n_out: int):
    info = plsc.get_sparse_core_info()
    nc, ns = info.num_cores, info.num_subcores
    nw = nc * ns
    assert n_out % nw == 0
    n_per_w = n_out // nw

    mesh = plsc.VectorSubcoreMesh(core_axis_name="c", subcore_axis_name="s")

    @functools.partial(
        pl.kernel,
        mesh=mesh,
        out_type=jax.ShapeDtypeStruct((n_out,), jnp.float32),
        scratch_types=[
            pltpu.VMEM((n_per_w,), jnp.int32),
            pltpu.VMEM((n_per_w,), jnp.float32),
            pltpu.SemaphoreType.DMA,
        ],
    )
    def gather_k(s_hbm, idx_hbm, out_hbm, idx_v, val_v, sem):
        wid = lax.axis_index("s") * nc + lax.axis_index("c")
        base = wid * n_per_w
        pltpu.sync_copy(idx_hbm.at[pl.ds(base, n_per_w)], idx_v)
        pltpu.async_copy(s_hbm.at[idx_v], val_v, sem).wait()
        pltpu.sync_copy(val_v, out_hbm.at[pl.ds(base, n_per_w)])

    return gather_k


def kernel(x, table, W, b):
    bsz, hist = x.shape
    n_out = bsz * hist
    s = _scores(table, W, b).reshape(-1)
    idx = x.reshape(-1).astype(jnp.int32)
    y = _make_gather(n_out)(s, idx)
    return y.reshape(bsz, hist, 1)
